# baseline (speedup 1.0000x reference)
import functools
import math

import jax
import jax.numpy as jnp
from jax import lax
from jax.experimental import pallas as pl
from jax.experimental.pallas import tpu as pltpu
from jax.experimental.pallas import tpu_sc as plsc

_VOCAB = 100000
_DIM = 128
_NUM_SAMPLED = 64
_B = 4096
_LOG_V1 = math.log(_VOCAB + 1.0)

_NC = 2
_NS = 16
_NW = _NC * _NS
_BPW = _B // _NW
_SW_WORKERS = 8
_SPW = _NUM_SAMPLED // _SW_WORKERS


def _sc_gather_body(emb_t, sc_t, sb_t, in_idx, t_idx, s_idx,
                    emb_o, tw_o, tb_o, sw_o, sb_o,
                    eidx_v, tidx_v, sidx_v, erows_v, trows_v, tb_v,
                    srows_v, sb_v, sem0, sem1, sem2, sem3, sem4,
                    sem5, sem6, sem7):
    wid = lax.axis_index("s") * _NC + lax.axis_index("c")
    base = wid * _BPW
    ci = pltpu.async_copy(in_idx.at[pl.ds(base, _BPW)], eidx_v, sem0)
    cj = pltpu.async_copy(t_idx.at[pl.ds(base, _BPW)], tidx_v, sem1)
    ci.wait()
    ce = pltpu.async_copy(emb_t.at[eidx_v], erows_v, sem0)
    cj.wait()
    ct = pltpu.async_copy(sc_t.at[tidx_v], trows_v, sem1)
    cb = pltpu.async_copy(sb_t.at[tidx_v], tb_v, sem2)

    @pl.when(wid < _SW_WORKERS)
    def _():
        sbase = wid * _SPW
        pltpu.async_copy(s_idx.at[pl.ds(sbase, _SPW)], sidx_v, sem3).wait()
        cs = pltpu.async_copy(sc_t.at[sidx_v], srows_v, sem3)
        cz = pltpu.async_copy(sb_t.at[sidx_v], sb_v, sem4)
        cs.wait()
        ws = pltpu.async_copy(srows_v, sw_o.at[pl.ds(sbase, _SPW)], sem5)
        cz.wait()
        wz = pltpu.async_copy(sb_v, sb_o.at[pl.ds(sbase, _SPW)], sem6)
        ws.wait()
        wz.wait()

    ce.wait()
    we = pltpu.async_copy(erows_v, emb_o.at[pl.ds(base, _BPW)], sem0)
    ct.wait()
    wt = pltpu.async_copy(trows_v, tw_o.at[pl.ds(base, _BPW)], sem1)
    cb.wait()
    wb = pltpu.async_copy(tb_v, tb_o.at[pl.ds(base, _BPW)], sem7)
    we.wait()
    wt.wait()
    wb.wait()


@functools.cache
def _make_sc_gather():
  return functools.partial(
    pl.kernel,
    out_type=(
        jax.ShapeDtypeStruct((_B, _DIM), jnp.float32),
        jax.ShapeDtypeStruct((_B, _DIM), jnp.float32),
        jax.ShapeDtypeStruct((_B,), jnp.float32),
        jax.ShapeDtypeStruct((_NUM_SAMPLED, _DIM), jnp.float32),
        jax.ShapeDtypeStruct((_NUM_SAMPLED,), jnp.float32),
    ),
    mesh=plsc.VectorSubcoreMesh(core_axis_name="c", subcore_axis_name="s"),
    scratch_types=[
        pltpu.VMEM((_BPW,), jnp.int32),
        pltpu.VMEM((_BPW,), jnp.int32),
        pltpu.VMEM((_SPW,), jnp.int32),
        pltpu.VMEM((_BPW, _DIM), jnp.float32),
        pltpu.VMEM((_BPW, _DIM), jnp.float32),
        pltpu.VMEM((_BPW,), jnp.float32),
        pltpu.VMEM((_SPW, _DIM), jnp.float32),
        pltpu.VMEM((_SPW,), jnp.float32),
        pltpu.SemaphoreType.DMA,
        pltpu.SemaphoreType.DMA,
        pltpu.SemaphoreType.DMA,
        pltpu.SemaphoreType.DMA,
        pltpu.SemaphoreType.DMA,
        pltpu.SemaphoreType.DMA,
        pltpu.SemaphoreType.DMA,
        pltpu.SemaphoreType.DMA,
    ],
  )(_sc_gather_body)


_BB = 512


def _dense_body(emb_r, tw_r, tb_r, tid_r, sw_r, sid_r, sb_r, out_r):
    emb = emb_r[...]
    tw = tw_r[...]
    tb = tb_r[...]
    tid = tid_r[...].astype(jnp.float32)
    logq_t = jnp.log((jnp.log(tid + 2.0) - jnp.log(tid + 1.0)) / _LOG_V1)
    ones_d = jnp.ones((1, _DIM), jnp.float32)
    rd = lax.dot_general(ones_d, emb * tw, (((1,), (1,)), ((), ())),
                         preferred_element_type=jnp.float32)
    tl = rd.reshape(_BB) + tb - logq_t
    sid = sid_r[...].astype(jnp.float32)
    logq_s = jnp.log((jnp.log(sid + 2.0) - jnp.log(sid + 1.0)) / _LOG_V1)
    soff = sb_r[...] - logq_s
    slg = lax.dot_general(emb, sw_r[...], (((1,), (1,)), ((), ())),
                          preferred_element_type=jnp.float32)
    slg = slg + soff[None, :]
    loss_true = jnp.maximum(tl, 0.0) - tl + jnp.log1p(jnp.exp(-jnp.abs(tl)))
    samp_terms = jnp.maximum(slg, 0.0) + jnp.log1p(jnp.exp(-jnp.abs(slg)))
    ones_s = jnp.ones((1, _NUM_SAMPLED), jnp.float32)
    ls = lax.dot_general(ones_s, samp_terms, (((1,), (1,)), ((), ())),
                         preferred_element_type=jnp.float32,
                         precision=lax.Precision.HIGHEST)
    loss_samp = ls.reshape(_BB)
    out_r[...] = loss_true + loss_samp


_dense_in_specs = [
    pl.BlockSpec((_BB, _DIM), lambda i: (i, 0)),
    pl.BlockSpec((_BB, _DIM), lambda i: (i, 0)),
    pl.BlockSpec((_BB,), lambda i: (i,)),
    pl.BlockSpec((_BB,), lambda i: (i,)),
    pl.BlockSpec((_NUM_SAMPLED, _DIM), lambda i: (0, 0)),
    pl.BlockSpec((_NUM_SAMPLED,), lambda i: (0,)),
    pl.BlockSpec((_NUM_SAMPLED,), lambda i: (0,)),
]
_dense_out_specs = pl.BlockSpec((_BB,), lambda i: (i,))

_dense = pl.pallas_call(
    _dense_body,
    grid=(_B // _BB,),
    in_specs=_dense_in_specs,
    out_specs=_dense_out_specs,
    out_shape=jax.ShapeDtypeStruct((_B,), jnp.float32),
)


def kernel(inputs, target, embedding_weights, score_weights, score_bias):
    in_idx = inputs.astype(jnp.int32)
    t_idx = target.reshape(-1).astype(jnp.int32)
    skey = jax.random.key(42)
    u = jax.random.uniform(skey, (_NUM_SAMPLED,), dtype=jnp.float32)
    sampled = jnp.clip(
        (jnp.exp(u * _LOG_V1) - 1.0).astype(jnp.int32), 0, _VOCAB - 1)
    emb, tw, tb, sw, sb = _make_sc_gather()(
        embedding_weights, score_weights, score_bias, in_idx, t_idx, sampled)
    return _dense(emb, tw, tb, t_idx, sw, sampled, sb)


# speedup vs baseline: 1.0391x; 1.0391x over previous
import functools
import math

import jax
import jax.numpy as jnp
from jax import lax
from jax.experimental import pallas as pl
from jax.experimental.pallas import tpu as pltpu
from jax.experimental.pallas import tpu_sc as plsc

_VOCAB = 100000
_DIM = 128
_NUM_SAMPLED = 64
_B = 4096
_LOG_V1 = math.log(_VOCAB + 1.0)

_NC = 2
_NS = 16
_NW = _NC * _NS
_BPW = _B // _NW
_SW_WORKERS = 8
_SPW = _NUM_SAMPLED // _SW_WORKERS


def _sc_gather_body(emb_t, sc_t, sb_t, in_idx, t_idx, s_idx,
                    emb_o, tw_o, tb_o, sw_o, sb_o,
                    eidx_v, tidx_v, sidx_v, erows_v, trows_v, tb_v,
                    srows_v, sb_v, sem0, sem1, sem2, sem3, sem4,
                    sem5, sem6, sem7):
    wid = lax.axis_index("s") * _NC + lax.axis_index("c")
    base = wid * _BPW
    ci = pltpu.async_copy(in_idx.at[pl.ds(base, _BPW)], eidx_v, sem0)
    cj = pltpu.async_copy(t_idx.at[pl.ds(base, _BPW)], tidx_v, sem1)
    ci.wait()
    ce = pltpu.async_copy(emb_t.at[eidx_v], erows_v, sem0)
    cj.wait()
    ct = pltpu.async_copy(sc_t.at[tidx_v], trows_v, sem1)
    cb = pltpu.async_copy(sb_t.at[tidx_v], tb_v, sem2)

    @pl.when(wid < _SW_WORKERS)
    def _():
        sbase = wid * _SPW
        pltpu.async_copy(s_idx.at[pl.ds(sbase, _SPW)], sidx_v, sem3).wait()
        cs = pltpu.async_copy(sc_t.at[sidx_v], srows_v, sem3)
        cz = pltpu.async_copy(sb_t.at[sidx_v], sb_v, sem4)
        cs.wait()
        ws = pltpu.async_copy(srows_v, sw_o.at[pl.ds(sbase, _SPW)], sem5)
        cz.wait()
        wz = pltpu.async_copy(sb_v, sb_o.at[pl.ds(sbase, _SPW)], sem6)
        ws.wait()
        wz.wait()

    ce.wait()
    we = pltpu.async_copy(erows_v, emb_o.at[pl.ds(base, _BPW)], sem0)
    ct.wait()
    wt = pltpu.async_copy(trows_v, tw_o.at[pl.ds(base, _BPW)], sem1)
    cb.wait()
    wb = pltpu.async_copy(tb_v, tb_o.at[pl.ds(base, _BPW)], sem7)
    we.wait()
    wt.wait()
    wb.wait()


@functools.cache
def _make_sc_gather():
  return functools.partial(
    pl.kernel,
    out_type=(
        jax.ShapeDtypeStruct((_B, _DIM), jnp.float32),
        jax.ShapeDtypeStruct((_B, _DIM), jnp.float32),
        jax.ShapeDtypeStruct((_B,), jnp.float32),
        jax.ShapeDtypeStruct((_NUM_SAMPLED, _DIM), jnp.float32),
        jax.ShapeDtypeStruct((_NUM_SAMPLED,), jnp.float32),
    ),
    mesh=plsc.VectorSubcoreMesh(core_axis_name="c", subcore_axis_name="s"),
    scratch_types=[
        pltpu.VMEM((_BPW,), jnp.int32),
        pltpu.VMEM((_BPW,), jnp.int32),
        pltpu.VMEM((_SPW,), jnp.int32),
        pltpu.VMEM((_BPW, _DIM), jnp.float32),
        pltpu.VMEM((_BPW, _DIM), jnp.float32),
        pltpu.VMEM((_BPW,), jnp.float32),
        pltpu.VMEM((_SPW, _DIM), jnp.float32),
        pltpu.VMEM((_SPW,), jnp.float32),
        pltpu.SemaphoreType.DMA,
        pltpu.SemaphoreType.DMA,
        pltpu.SemaphoreType.DMA,
        pltpu.SemaphoreType.DMA,
        pltpu.SemaphoreType.DMA,
        pltpu.SemaphoreType.DMA,
        pltpu.SemaphoreType.DMA,
        pltpu.SemaphoreType.DMA,
    ],
  )(_sc_gather_body)


_BB = 512


def _dense_body(emb_r, tw_r, tb_r, tid_r, sw_r, sid_r, sb_r, out_r):
    emb = emb_r[...]
    tw = tw_r[...]
    tb = tb_r[...]
    tid = tid_r[...].astype(jnp.float32)
    logq_t = jnp.log((jnp.log(tid + 2.0) - jnp.log(tid + 1.0)) / _LOG_V1)
    ones_d = jnp.ones((1, _DIM), jnp.float32)
    rd = lax.dot_general(ones_d, emb * tw, (((1,), (1,)), ((), ())),
                         preferred_element_type=jnp.float32)
    tl = rd.reshape(_BB) + tb - logq_t
    sid = sid_r[...].astype(jnp.float32)
    logq_s = jnp.log((jnp.log(sid + 2.0) - jnp.log(sid + 1.0)) / _LOG_V1)
    soff = sb_r[...] - logq_s
    slg = lax.dot_general(emb, sw_r[...], (((1,), (1,)), ((), ())),
                          preferred_element_type=jnp.float32)
    slg = slg + soff[None, :]
    loss_true = jnp.maximum(tl, 0.0) - tl + jnp.log1p(jnp.exp(-jnp.abs(tl)))
    samp_terms = jnp.maximum(slg, 0.0) + jnp.log1p(jnp.exp(-jnp.abs(slg)))
    resid_terms = samp_terms - soff[None, :]
    soff_sum = jnp.sum(soff)
    ones_s = jnp.ones((1, _NUM_SAMPLED), jnp.float32)
    ls = lax.dot_general(ones_s, resid_terms, (((1,), (1,)), ((), ())),
                         preferred_element_type=jnp.float32)
    loss_samp = ls.reshape(_BB) + soff_sum
    out_r[...] = loss_true + loss_samp


_dense_in_specs = [
    pl.BlockSpec((_BB, _DIM), lambda i: (i, 0)),
    pl.BlockSpec((_BB, _DIM), lambda i: (i, 0)),
    pl.BlockSpec((_BB,), lambda i: (i,)),
    pl.BlockSpec((_BB,), lambda i: (i,)),
    pl.BlockSpec((_NUM_SAMPLED, _DIM), lambda i: (0, 0)),
    pl.BlockSpec((_NUM_SAMPLED,), lambda i: (0,)),
    pl.BlockSpec((_NUM_SAMPLED,), lambda i: (0,)),
]
_dense_out_specs = pl.BlockSpec((_BB,), lambda i: (i,))

_dense = pl.pallas_call(
    _dense_body,
    grid=(_B // _BB,),
    in_specs=_dense_in_specs,
    out_specs=_dense_out_specs,
    out_shape=jax.ShapeDtypeStruct((_B,), jnp.float32),
)


def kernel(inputs, target, embedding_weights, score_weights, score_bias):
    in_idx = inputs.astype(jnp.int32)
    t_idx = target.reshape(-1).astype(jnp.int32)
    skey = jax.random.key(42)
    u = jax.random.uniform(skey, (_NUM_SAMPLED,), dtype=jnp.float32)
    sampled = jnp.clip(
        (jnp.exp(u * _LOG_V1) - 1.0).astype(jnp.int32), 0, _VOCAB - 1)
    emb, tw, tb, sw, sb = _make_sc_gather()(
        embedding_weights, score_weights, score_bias, in_idx, t_idx, sampled)
    return _dense(emb, tw, tb, t_idx, sw, sampled, sb)


# speedup vs baseline: 1.1180x; 1.0759x over previous
import functools
import math

import jax
import jax.numpy as jnp
from jax import lax
from jax.experimental import pallas as pl
from jax.experimental.pallas import tpu as pltpu
from jax.experimental.pallas import tpu_sc as plsc

_VOCAB = 100000
_DIM = 128
_NUM_SAMPLED = 64
_B = 4096
_LOG_V1 = math.log(_VOCAB + 1.0)

_NC = 2
_NS = 16
_NW = _NC * _NS
_BPW = _B // _NW
_SW_WORKERS = 8
_SPW = _NUM_SAMPLED // _SW_WORKERS


def _sc_gather_body(emb_t, sc_t, sb_t, in_idx, t_idx, s_idx,
                    emb_o, tw_o, tb_o, sw_o, sb_o,
                    eidx_v, tidx_v, sidx_v, erows_v, trows_v, tb_v,
                    srows_v, sb_v, sem0, sem1, sem2, sem3, sem4,
                    sem5, sem6, sem7):
    wid = lax.axis_index("s") * _NC + lax.axis_index("c")
    base = wid * _BPW
    ci = pltpu.async_copy(in_idx.at[pl.ds(base, _BPW)], eidx_v, sem0)
    cj = pltpu.async_copy(t_idx.at[pl.ds(base, _BPW)], tidx_v, sem1)
    ci.wait()
    ce = pltpu.async_copy(emb_t.at[eidx_v], erows_v, sem0)
    cj.wait()
    ct = pltpu.async_copy(sc_t.at[tidx_v], trows_v, sem1)
    cb = pltpu.async_copy(sb_t.at[tidx_v], tb_v, sem2)

    @pl.when(wid < _SW_WORKERS)
    def _():
        sbase = wid * _SPW
        pltpu.async_copy(s_idx.at[pl.ds(sbase, _SPW)], sidx_v, sem3).wait()
        cs = pltpu.async_copy(sc_t.at[sidx_v], srows_v, sem3)
        cz = pltpu.async_copy(sb_t.at[sidx_v], sb_v, sem4)
        cs.wait()
        ws = pltpu.async_copy(srows_v, sw_o.at[pl.ds(sbase, _SPW)], sem5)
        cz.wait()
        wz = pltpu.async_copy(sb_v, sb_o.at[pl.ds(sbase, _SPW)], sem6)
        ws.wait()
        wz.wait()

    ce.wait()
    we = pltpu.async_copy(erows_v, emb_o.at[pl.ds(base, _BPW)], sem0)
    ct.wait()
    wt = pltpu.async_copy(trows_v, tw_o.at[pl.ds(base, _BPW)], sem1)
    cb.wait()
    wb = pltpu.async_copy(tb_v, tb_o.at[pl.ds(base, _BPW)], sem7)
    we.wait()
    wt.wait()
    wb.wait()


@functools.cache
def _make_sc_gather():
  return functools.partial(
    pl.kernel,
    out_type=(
        jax.ShapeDtypeStruct((_B, _DIM), jnp.float32),
        jax.ShapeDtypeStruct((_B, _DIM), jnp.float32),
        jax.ShapeDtypeStruct((_B,), jnp.float32),
        jax.ShapeDtypeStruct((_NUM_SAMPLED, _DIM), jnp.float32),
        jax.ShapeDtypeStruct((_NUM_SAMPLED,), jnp.float32),
    ),
    mesh=plsc.VectorSubcoreMesh(core_axis_name="c", subcore_axis_name="s"),
    scratch_types=[
        pltpu.VMEM((_BPW,), jnp.int32),
        pltpu.VMEM((_BPW,), jnp.int32),
        pltpu.VMEM((_SPW,), jnp.int32),
        pltpu.VMEM((_BPW, _DIM), jnp.float32),
        pltpu.VMEM((_BPW, _DIM), jnp.float32),
        pltpu.VMEM((_BPW,), jnp.float32),
        pltpu.VMEM((_SPW, _DIM), jnp.float32),
        pltpu.VMEM((_SPW,), jnp.float32),
        pltpu.SemaphoreType.DMA,
        pltpu.SemaphoreType.DMA,
        pltpu.SemaphoreType.DMA,
        pltpu.SemaphoreType.DMA,
        pltpu.SemaphoreType.DMA,
        pltpu.SemaphoreType.DMA,
        pltpu.SemaphoreType.DMA,
        pltpu.SemaphoreType.DMA,
    ],
  )(_sc_gather_body)


_BB = 1024


def _dense_body(emb_r, tw_r, tb_r, tid_r, sw_r, sid_r, sb_r, out_r):
    emb = emb_r[...]
    tw = tw_r[...]
    tb = tb_r[...]
    tid = tid_r[...].astype(jnp.float32)
    logq_t = jnp.log((jnp.log(tid + 2.0) - jnp.log(tid + 1.0)) / _LOG_V1)
    ones_d = jnp.ones((1, _DIM), jnp.float32)
    rd = lax.dot_general(ones_d, emb * tw, (((1,), (1,)), ((), ())),
                         preferred_element_type=jnp.float32)
    tl = rd.reshape(_BB) + tb - logq_t
    sid = sid_r[...].astype(jnp.float32)
    logq_s = jnp.log((jnp.log(sid + 2.0) - jnp.log(sid + 1.0)) / _LOG_V1)
    soff = sb_r[...] - logq_s
    slg = lax.dot_general(emb, sw_r[...], (((1,), (1,)), ((), ())),
                          preferred_element_type=jnp.float32)
    slg = slg + soff[None, :]
    loss_true = jnp.maximum(tl, 0.0) - tl + jnp.log1p(jnp.exp(-jnp.abs(tl)))
    samp_terms = jnp.maximum(slg, 0.0) + jnp.log1p(jnp.exp(-jnp.abs(slg)))
    resid_terms = samp_terms - soff[None, :]
    soff_sum = jnp.sum(soff)
    ones_s = jnp.ones((1, _NUM_SAMPLED), jnp.float32)
    ls = lax.dot_general(ones_s, resid_terms, (((1,), (1,)), ((), ())),
                         preferred_element_type=jnp.float32)
    loss_samp = ls.reshape(_BB) + soff_sum
    out_r[...] = loss_true + loss_samp


_dense_in_specs = [
    pl.BlockSpec((_BB, _DIM), lambda i: (i, 0)),
    pl.BlockSpec((_BB, _DIM), lambda i: (i, 0)),
    pl.BlockSpec((_BB,), lambda i: (i,)),
    pl.BlockSpec((_BB,), lambda i: (i,)),
    pl.BlockSpec((_NUM_SAMPLED, _DIM), lambda i: (0, 0)),
    pl.BlockSpec((_NUM_SAMPLED,), lambda i: (0,)),
    pl.BlockSpec((_NUM_SAMPLED,), lambda i: (0,)),
]
_dense_out_specs = pl.BlockSpec((_BB,), lambda i: (i,))

_dense = pl.pallas_call(
    _dense_body,
    grid=(_B // _BB,),
    in_specs=_dense_in_specs,
    out_specs=_dense_out_specs,
    out_shape=jax.ShapeDtypeStruct((_B,), jnp.float32),
)


def kernel(inputs, target, embedding_weights, score_weights, score_bias):
    in_idx = inputs.astype(jnp.int32)
    t_idx = target.reshape(-1).astype(jnp.int32)
    skey = jax.random.key(42)
    u = jax.random.uniform(skey, (_NUM_SAMPLED,), dtype=jnp.float32)
    sampled = jnp.clip(
        (jnp.exp(u * _LOG_V1) - 1.0).astype(jnp.int32), 0, _VOCAB - 1)
    emb, tw, tb, sw, sb = _make_sc_gather()(
        embedding_weights, score_weights, score_bias, in_idx, t_idx, sampled)
    return _dense(emb, tw, tb, t_idx, sw, sampled, sb)


# speedup vs baseline: 1.1662x; 1.0431x over previous
import functools
import math

import jax
import jax.numpy as jnp
from jax import lax
from jax.experimental import pallas as pl
from jax.experimental.pallas import tpu as pltpu
from jax.experimental.pallas import tpu_sc as plsc

_VOCAB = 100000
_DIM = 128
_NUM_SAMPLED = 64
_B = 4096
_LOG_V1 = math.log(_VOCAB + 1.0)

_NC = 2
_NS = 16
_NW = _NC * _NS
_BPW = _B // _NW
_SW_WORKERS = 8
_SPW = _NUM_SAMPLED // _SW_WORKERS


def _sc_gather_body(emb_t, sc_t, sb_t, in_idx, t_idx, s_idx,
                    emb_o, tw_o, tb_o, sw_o, sb_o,
                    eidx_v, tidx_v, sidx_v, erows_v, trows_v, tb_v,
                    srows_v, sb_v, sem0, sem1, sem2, sem3, sem4,
                    sem5, sem6, sem7):
    wid = lax.axis_index("s") * _NC + lax.axis_index("c")
    base = wid * _BPW
    ci = pltpu.async_copy(in_idx.at[pl.ds(base, _BPW)], eidx_v, sem0)
    cj = pltpu.async_copy(t_idx.at[pl.ds(base, _BPW)], tidx_v, sem1)
    ci.wait()
    ce = pltpu.async_copy(emb_t.at[eidx_v], erows_v, sem0)
    cj.wait()
    ct = pltpu.async_copy(sc_t.at[tidx_v], trows_v, sem1)
    cb = pltpu.async_copy(sb_t.at[tidx_v], tb_v, sem2)

    @pl.when(wid < _SW_WORKERS)
    def _():
        sbase = wid * _SPW
        pltpu.async_copy(s_idx.at[pl.ds(sbase, _SPW)], sidx_v, sem3).wait()
        cs = pltpu.async_copy(sc_t.at[sidx_v], srows_v, sem3)
        cz = pltpu.async_copy(sb_t.at[sidx_v], sb_v, sem4)
        cs.wait()
        ws = pltpu.async_copy(srows_v, sw_o.at[pl.ds(sbase, _SPW)], sem5)
        cz.wait()
        wz = pltpu.async_copy(sb_v, sb_o.at[pl.ds(sbase, _SPW)], sem6)
        ws.wait()
        wz.wait()

    ce.wait()
    we = pltpu.async_copy(erows_v, emb_o.at[pl.ds(base, _BPW)], sem0)
    ct.wait()
    wt = pltpu.async_copy(trows_v, tw_o.at[pl.ds(base, _BPW)], sem1)
    cb.wait()
    wb = pltpu.async_copy(tb_v, tb_o.at[pl.ds(base, _BPW)], sem7)
    we.wait()
    wt.wait()
    wb.wait()


@functools.cache
def _make_sc_gather():
  return functools.partial(
    pl.kernel,
    out_type=(
        jax.ShapeDtypeStruct((_B, _DIM), jnp.float32),
        jax.ShapeDtypeStruct((_B, _DIM), jnp.float32),
        jax.ShapeDtypeStruct((_B,), jnp.float32),
        jax.ShapeDtypeStruct((_NUM_SAMPLED, _DIM), jnp.float32),
        jax.ShapeDtypeStruct((_NUM_SAMPLED,), jnp.float32),
    ),
    mesh=plsc.VectorSubcoreMesh(core_axis_name="c", subcore_axis_name="s"),
    scratch_types=[
        pltpu.VMEM((_BPW,), jnp.int32),
        pltpu.VMEM((_BPW,), jnp.int32),
        pltpu.VMEM((_SPW,), jnp.int32),
        pltpu.VMEM((_BPW, _DIM), jnp.float32),
        pltpu.VMEM((_BPW, _DIM), jnp.float32),
        pltpu.VMEM((_BPW,), jnp.float32),
        pltpu.VMEM((_SPW, _DIM), jnp.float32),
        pltpu.VMEM((_SPW,), jnp.float32),
        pltpu.SemaphoreType.DMA,
        pltpu.SemaphoreType.DMA,
        pltpu.SemaphoreType.DMA,
        pltpu.SemaphoreType.DMA,
        pltpu.SemaphoreType.DMA,
        pltpu.SemaphoreType.DMA,
        pltpu.SemaphoreType.DMA,
        pltpu.SemaphoreType.DMA,
    ],
  )(_sc_gather_body)


_BB = 2048


def _dense_body(emb_r, tw_r, tb_r, tid_r, sw_r, sid_r, sb_r, out_r):
    emb = emb_r[...]
    tw = tw_r[...]
    tb = tb_r[...]
    tid = tid_r[...].astype(jnp.float32)
    logq_t = jnp.log((jnp.log(tid + 2.0) - jnp.log(tid + 1.0)) / _LOG_V1)
    ones_d = jnp.ones((1, _DIM), jnp.float32)
    rd = lax.dot_general(ones_d, emb * tw, (((1,), (1,)), ((), ())),
                         preferred_element_type=jnp.float32)
    tl = rd.reshape(_BB) + tb - logq_t
    sid = sid_r[...].astype(jnp.float32)
    logq_s = jnp.log((jnp.log(sid + 2.0) - jnp.log(sid + 1.0)) / _LOG_V1)
    soff = sb_r[...] - logq_s
    slg = lax.dot_general(emb, sw_r[...], (((1,), (1,)), ((), ())),
                          preferred_element_type=jnp.float32)
    slg = slg + soff[None, :]
    loss_true = jnp.maximum(tl, 0.0) - tl + jnp.log(1.0 + jnp.exp(-jnp.abs(tl)))
    samp_terms = jnp.maximum(slg, 0.0) + jnp.log(1.0 + jnp.exp(-jnp.abs(slg)))
    resid_terms = samp_terms - soff[None, :]
    soff_sum = jnp.sum(soff)
    ones_s = jnp.ones((1, _NUM_SAMPLED), jnp.float32)
    ls = lax.dot_general(ones_s, resid_terms, (((1,), (1,)), ((), ())),
                         preferred_element_type=jnp.float32)
    loss_samp = ls.reshape(_BB) + soff_sum
    out_r[...] = loss_true + loss_samp


_dense_in_specs = [
    pl.BlockSpec((_BB, _DIM), lambda i: (i, 0)),
    pl.BlockSpec((_BB, _DIM), lambda i: (i, 0)),
    pl.BlockSpec((_BB,), lambda i: (i,)),
    pl.BlockSpec((_BB,), lambda i: (i,)),
    pl.BlockSpec((_NUM_SAMPLED, _DIM), lambda i: (0, 0)),
    pl.BlockSpec((_NUM_SAMPLED,), lambda i: (0,)),
    pl.BlockSpec((_NUM_SAMPLED,), lambda i: (0,)),
]
_dense_out_specs = pl.BlockSpec((_BB,), lambda i: (i,))

_dense = pl.pallas_call(
    _dense_body,
    grid=(_B // _BB,),
    in_specs=_dense_in_specs,
    out_specs=_dense_out_specs,
    out_shape=jax.ShapeDtypeStruct((_B,), jnp.float32),
)


def kernel(inputs, target, embedding_weights, score_weights, score_bias):
    in_idx = inputs.astype(jnp.int32)
    t_idx = target.reshape(-1).astype(jnp.int32)
    skey = jax.random.key(42)
    u = jax.random.uniform(skey, (_NUM_SAMPLED,), dtype=jnp.float32)
    sampled = jnp.clip(
        (jnp.exp(u * _LOG_V1) - 1.0).astype(jnp.int32), 0, _VOCAB - 1)
    emb, tw, tb, sw, sb = _make_sc_gather()(
        embedding_weights, score_weights, score_bias, in_idx, t_idx, sampled)
    return _dense(emb, tw, tb, t_idx, sw, sampled, sb)
